# Initial kernel scaffold; baseline (speedup 1.0000x reference)
import jax
import jax.numpy as jnp
from jax.experimental import pallas as pl


def kernel(x, emb, W1, b1, g1, be1, W2, b2, g2, be2, W3, b3):
    raise NotImplementedError("write your pallas kernel here")


# speedup vs baseline: 22.9979x; 22.9979x over previous
import dataclasses
import functools

import numpy as np
import jax
import jax.numpy as jnp
from jax.experimental import pallas as pl
from jax.experimental.pallas import tpu as pltpu
from jax.experimental.pallas import tpu_sc as plsc

F = 26
FD = 4000
D = 16
B = 16384
NPAIR = F * (F - 1) // 2
KPAD = 768
NCHUNK = KPAD // 128
NLEFT = 26
NRIGHT = 351
NGROUPS = 21
INPUT_DIM = F * D + NPAIR

NC, NS = 2, 16
NW = NC * NS
SPW = B // NW
G = 4

_fsel = np.zeros(KPAD, np.int32)
_cbase = np.zeros(KPAD, np.int32)
for _f in range(F):
    _fsel[_f] = _f
    _cbase[_f] = FD * _f
_p = 0
for _i in range(F - 1):
    for _j in range(_i + 1, F):
        _fsel[NLEFT + _p] = _i
        _cbase[NLEFT + _p] = _j * F * FD + FD * _i
        _fsel[NRIGHT + _p] = _j
        _cbase[NRIGHT + _p] = (_i + 1) * F * FD + FD * _j
        _p += 1
FSEL = _fsel
CBASE = _cbase


def _sc_body(idx_hbm, emb_hbm, out_hbm, idx_v, rows_v, out_v, sem):
    wid = jax.lax.axis_index("s") * NC + jax.lax.axis_index("c")
    base = wid * SPW
    lane = jax.lax.iota(jnp.int32, 16)

    @pl.loop(0, SPW // G)
    def _(it):
        s0 = base + it * G
        pltpu.sync_copy(idx_hbm.at[pl.ds(s0, G)], idx_v)
        copies = []
        for sb in range(G):
            for c in range(NCHUNK):
                copies.append(pltpu.async_copy(
                    emb_hbm.at[idx_v.at[sb, c]],
                    rows_v.at[sb, pl.ds(c * 128, 128)], sem))
        for cp in copies:
            cp.wait()
        for sb in range(G):
            sbv = jnp.full((16,), sb, jnp.int32)

            @pl.loop(0, F)
            def _(cc):
                out_v[sb, pl.ds(cc * D, D)] = rows_v[sb, cc, :]

            @pl.loop(0, NGROUPS)
            def _(g):
                ra = NLEFT + g * 16 + lane
                rb = NRIGHT + g * 16 + lane
                acc = jnp.zeros((16,), jnp.float32)
                for d in range(D):
                    dv = jnp.full((16,), d, jnp.int32)
                    a = plsc.load_gather(rows_v, [sbv, ra, dv])
                    b = plsc.load_gather(rows_v, [sbv, rb, dv])
                    acc = acc + a * b
                out_v[sb, pl.ds(F * D + g * 16, 16)] = acc

        pltpu.sync_copy(out_v, out_hbm.at[pl.ds(s0, G)])


def _sc_gather_ffm(embflat, idx):
    mesh = plsc.VectorSubcoreMesh(
        core_axis_name="c", subcore_axis_name="s", num_cores=NC,
        num_subcores=NS)
    cp = pltpu.CompilerParams(use_tc_tiling_on_sc=False)
    if "needs_layout_passes" in pltpu.CompilerParams.__dataclass_fields__:
        cp = dataclasses.replace(cp, needs_layout_passes=False)
    fn = pl.kernel(
        _sc_body,
        out_type=jax.ShapeDtypeStruct((B, KPAD), jnp.float32),
        mesh=mesh,
        compiler_params=cp,
        scratch_types=[
            pltpu.VMEM((G, NCHUNK, 128), jnp.int32),
            pltpu.VMEM((G, KPAD, D), jnp.float32),
            pltpu.VMEM((G, KPAD), jnp.float32),
            pltpu.SemaphoreType.DMA,
        ],
    )
    return fn(idx, embflat)


TB = 2048
NT = B // TB


def _mlp_body(dnn_ref, W1_ref, b1_ref, g1_ref, be1_ref, W2_ref, b2_ref,
              g2_ref, be2_ref, W3_ref, b3_ref, out_ref,
              h1_ref, h2_ref, s1_ref, q1_ref, s2_ref, q2_ref):
    p = pl.program_id(0)
    t = pl.program_id(1)
    inv_b = jnp.float32(1.0 / B)

    @pl.when(p == 0)
    def _():
        @pl.when(t == 0)
        def _():
            s1_ref[...] = jnp.zeros_like(s1_ref)
            q1_ref[...] = jnp.zeros_like(q1_ref)
            s2_ref[...] = jnp.zeros_like(s2_ref)
            q2_ref[...] = jnp.zeros_like(q2_ref)

        h1 = jnp.dot(dnn_ref[...], W1_ref[...],
                     preferred_element_type=jnp.float32) + b1_ref[...]
        h1_ref[pl.ds(t * TB, TB), :] = h1
        s1_ref[...] += jnp.sum(h1, axis=0, keepdims=True)
        q1_ref[...] += jnp.sum(h1 * h1, axis=0, keepdims=True)

    @pl.when(p == 1)
    def _():
        m1 = s1_ref[...] * inv_b
        v1 = q1_ref[...] * inv_b - m1 * m1
        inv1 = jax.lax.rsqrt(v1 + 1e-5)
        h1 = h1_ref[pl.ds(t * TB, TB), :]
        a1 = jnp.maximum((h1 - m1) * inv1 * g1_ref[...] + be1_ref[...], 0.0)
        h2 = jnp.dot(a1, W2_ref[...],
                     preferred_element_type=jnp.float32) + b2_ref[...]
        h2_ref[pl.ds(t * TB, TB), :] = h2
        s2_ref[...] += jnp.sum(h2, axis=0, keepdims=True)
        q2_ref[...] += jnp.sum(h2 * h2, axis=0, keepdims=True)

    @pl.when(p == 2)
    def _():
        m2 = s2_ref[...] * inv_b
        v2 = q2_ref[...] * inv_b - m2 * m2
        inv2 = jax.lax.rsqrt(v2 + 1e-5)
        h2 = h2_ref[pl.ds(t * TB, TB), :]
        a2 = jnp.maximum((h2 - m2) * inv2 * g2_ref[...] + be2_ref[...], 0.0)
        y = jnp.sum(a2 * W3_ref[...], axis=1, keepdims=True) + b3_ref[...]
        out_ref[...] = jax.nn.sigmoid(y)


def _tc_mlp(dnn, W1p, b1, g1, be1, W2, b2, g2, be2, W3, b3):
    row = lambda v: v.reshape(1, -1)
    grid = (3, NT)
    out = pl.pallas_call(
        _mlp_body,
        grid=grid,
        in_specs=[
            pl.BlockSpec((TB, KPAD), lambda p, t: (jnp.where(p == 0, t, 0), 0)),
            pl.BlockSpec((KPAD, 64), lambda p, t: (0, 0)),
            pl.BlockSpec((1, 64), lambda p, t: (0, 0)),
            pl.BlockSpec((1, 64), lambda p, t: (0, 0)),
            pl.BlockSpec((1, 64), lambda p, t: (0, 0)),
            pl.BlockSpec((64, 32), lambda p, t: (0, 0)),
            pl.BlockSpec((1, 32), lambda p, t: (0, 0)),
            pl.BlockSpec((1, 32), lambda p, t: (0, 0)),
            pl.BlockSpec((1, 32), lambda p, t: (0, 0)),
            pl.BlockSpec((1, 32), lambda p, t: (0, 0)),
            pl.BlockSpec((1, 1), lambda p, t: (0, 0)),
        ],
        out_specs=pl.BlockSpec((TB, 1), lambda p, t: (t, 0)),
        out_shape=jax.ShapeDtypeStruct((B, 1), jnp.float32),
        scratch_shapes=[
            pltpu.VMEM((B, 64), jnp.float32),
            pltpu.VMEM((B, 32), jnp.float32),
            pltpu.VMEM((1, 64), jnp.float32),
            pltpu.VMEM((1, 64), jnp.float32),
            pltpu.VMEM((1, 32), jnp.float32),
            pltpu.VMEM((1, 32), jnp.float32),
        ],
    )(dnn, W1p, row(b1), row(g1), row(be1), W2, row(b2), row(g2), row(be2),
      W3.reshape(1, -1), b3.reshape(1, 1))
    return out


def kernel(x, emb, W1, b1, g1, be1, W2, b2, g2, be2, W3, b3):
    embflat = emb.reshape(F * F * FD, D)
    idx = (jnp.take(x, jnp.asarray(FSEL), axis=1)
           + jnp.asarray(CBASE)[None, :]).astype(jnp.int32)
    idx = idx.reshape(B, NCHUNK, 128)
    dnn = _sc_gather_ffm(embflat, idx)
    W1p = jnp.concatenate(
        [W1, jnp.zeros((KPAD - INPUT_DIM, 64), jnp.float32)], axis=0)
    y = _tc_mlp(dnn, W1p, b1, g1, be1, W2, b2, g2, be2, W3, b3)
    return jnp.squeeze(y, axis=1)


# speedup vs baseline: 28.1222x; 1.2228x over previous
import dataclasses
import functools

import numpy as np
import jax
import jax.numpy as jnp
from jax.experimental import pallas as pl
from jax.experimental.pallas import tpu as pltpu
from jax.experimental.pallas import tpu_sc as plsc

F = 26
FD = 4000
D = 16
B = 16384
NPAIR = F * (F - 1) // 2
KPAD = 768
NROWS = 688
NLEFT = 26
NRIGHT = 351
NGROUPS = 21
INPUT_DIM = F * D + NPAIR

NC, NS = 2, 16
NW = NC * NS
SPW = B // NW
G = 4

_fsel = np.zeros(KPAD, np.int32)
_cbase = np.zeros(KPAD, np.int32)
for _f in range(F):
    _fsel[_f] = _f
    _cbase[_f] = FD * _f
_p = 0
for _i in range(F - 1):
    for _j in range(_i + 1, F):
        _fsel[NLEFT + _p] = _i
        _cbase[NLEFT + _p] = _j * F * FD + FD * _i
        _fsel[NRIGHT + _p] = _j
        _cbase[NRIGHT + _p] = (_i + 1) * F * FD + FD * _j
        _p += 1
FSEL = _fsel
CBASE = _cbase


NIT = SPW // G
_CHUNKS = [(c * 128, 128) for c in range(5)] + [(640, 48)]


def _sc_body(x_hbm, fsel_hbm, cbase_hbm, emb_hbm, out_hbm,
             xq0, xq1, idx0, idx1, rows0, rows1, out0, out1,
             fselv, cbasev, gsem0, gsem1, osem0, osem1):
    wid = jax.lax.axis_index("s") * NC + jax.lax.axis_index("c")
    base = wid * SPW
    lane = jax.lax.iota(jnp.int32, 16)
    bufs = ((xq0, idx0, rows0, gsem0, out0, osem0),
            (xq1, idx1, rows1, gsem1, out1, osem1))

    pltpu.sync_copy(fsel_hbm, fselv)
    pltpu.sync_copy(cbase_hbm, cbasev)
    zeros16 = jnp.zeros((16,), jnp.float32)
    for bf in bufs:
        for sb in range(G):
            bf[4][sb, pl.ds(752, 16)] = zeros16

    def fire(i, xq, idxv, rows, gsem):
        s0 = base + i * G
        pltpu.sync_copy(x_hbm.at[pl.ds(s0, G)], xq)
        for sb in range(G):
            sbv = jnp.full((16,), sb, jnp.int32)

            @pl.loop(0, NROWS // 16)
            def _(kk):
                fv = fselv[pl.ds(kk * 16, 16)]
                cb = cbasev[pl.ds(kk * 16, 16)]
                xv = plsc.load_gather(xq, [sbv, fv])
                idxv[sb, pl.ds(kk * 16, 16)] = xv + cb

        for sb in range(G):
            for off, ln in _CHUNKS:
                pltpu.async_copy(
                    emb_hbm.at[idxv.at[sb, pl.ds(off, ln)]],
                    rows.at[sb, pl.ds(off, ln)], gsem)

    def drain_gathers(idxv, rows, gsem):
        for sb in range(G):
            for off, ln in _CHUNKS:
                pltpu.make_async_copy(
                    emb_hbm.at[idxv.at[sb, pl.ds(off, ln)]],
                    rows.at[sb, pl.ds(off, ln)], gsem).wait()

    def drain_out(i, outv, osem):
        s0 = base + i * G
        pltpu.make_async_copy(outv, out_hbm.at[pl.ds(s0, G)], osem).wait()

    def compute(i, rows, outv, osem):
        for sb in range(G):
            sbv = jnp.full((16,), sb, jnp.int32)

            @pl.loop(0, F)
            def _(cc):
                outv[sb, pl.ds(cc * D, D)] = rows[sb, cc, :]

            @pl.loop(0, NGROUPS)
            def _(g):
                ra = NLEFT + g * 16 + lane
                rb = NRIGHT + g * 16 + lane
                acc = jnp.zeros((16,), jnp.float32)
                for d in range(D):
                    dv = jnp.full((16,), d, jnp.int32)
                    a = plsc.load_gather(rows, [sbv, ra, dv])
                    b = plsc.load_gather(rows, [sbv, rb, dv])
                    acc = acc + a * b
                outv[sb, pl.ds(F * D + g * 16, 16)] = acc
        s0 = base + i * G
        pltpu.async_copy(outv, out_hbm.at[pl.ds(s0, G)], osem)

    fire(0, bufs[0][0], bufs[0][1], bufs[0][2], bufs[0][3])

    @pl.loop(0, NIT, step=2)
    def _(it):
        for p in range(2):
            i = it + p
            xq_n, idx_n, rows_n, gsem_n = bufs[1 - p][:4]
            xq_c, idx_c, rows_c, gsem_c, out_c, osem_c = bufs[p]

            @pl.when(i + 1 < NIT)
            def _():
                fire(i + 1, xq_n, idx_n, rows_n, gsem_n)

            drain_gathers(idx_c, rows_c, gsem_c)

            @pl.when(i >= 2)
            def _():
                drain_out(i, out_c, osem_c)

            compute(i, rows_c, out_c, osem_c)

    drain_out(NIT - 2, bufs[0][4], bufs[0][5])
    drain_out(NIT - 1, bufs[1][4], bufs[1][5])


def _sc_gather_ffm(xpad, fsel, cbase, embflat):
    mesh = plsc.VectorSubcoreMesh(
        core_axis_name="c", subcore_axis_name="s", num_cores=NC,
        num_subcores=NS)
    cp = pltpu.CompilerParams(use_tc_tiling_on_sc=False)
    if "needs_layout_passes" in pltpu.CompilerParams.__dataclass_fields__:
        cp = dataclasses.replace(cp, needs_layout_passes=False)
    fn = pl.kernel(
        _sc_body,
        out_type=jax.ShapeDtypeStruct((B, KPAD), jnp.float32),
        mesh=mesh,
        compiler_params=cp,
        scratch_types=[
            pltpu.VMEM((G, 32), jnp.int32),
            pltpu.VMEM((G, 32), jnp.int32),
            pltpu.VMEM((G, KPAD), jnp.int32),
            pltpu.VMEM((G, KPAD), jnp.int32),
            pltpu.VMEM((G, NROWS, D), jnp.float32),
            pltpu.VMEM((G, NROWS, D), jnp.float32),
            pltpu.VMEM((G, KPAD), jnp.float32),
            pltpu.VMEM((G, KPAD), jnp.float32),
            pltpu.VMEM((KPAD,), jnp.int32),
            pltpu.VMEM((KPAD,), jnp.int32),
            pltpu.SemaphoreType.DMA,
            pltpu.SemaphoreType.DMA,
            pltpu.SemaphoreType.DMA,
            pltpu.SemaphoreType.DMA,
        ],
    )
    return fn(xpad, fsel, cbase, embflat)


TB = 2048
NT = B // TB


def _mlp_body(dnn_ref, W1_ref, b1_ref, g1_ref, be1_ref, W2_ref, b2_ref,
              g2_ref, be2_ref, W3_ref, b3_ref, out_ref,
              h1_ref, h2_ref, s1_ref, q1_ref, s2_ref, q2_ref):
    p = pl.program_id(0)
    t = pl.program_id(1)
    inv_b = jnp.float32(1.0 / B)

    @pl.when(p == 0)
    def _():
        @pl.when(t == 0)
        def _():
            s1_ref[...] = jnp.zeros_like(s1_ref)
            q1_ref[...] = jnp.zeros_like(q1_ref)
            s2_ref[...] = jnp.zeros_like(s2_ref)
            q2_ref[...] = jnp.zeros_like(q2_ref)

        h1 = jnp.dot(dnn_ref[...], W1_ref[...],
                     preferred_element_type=jnp.float32) + b1_ref[...]
        h1_ref[pl.ds(t * TB, TB), :] = h1
        s1_ref[...] += jnp.sum(h1, axis=0, keepdims=True)
        q1_ref[...] += jnp.sum(h1 * h1, axis=0, keepdims=True)

    @pl.when(p == 1)
    def _():
        m1 = s1_ref[...] * inv_b
        v1 = q1_ref[...] * inv_b - m1 * m1
        inv1 = jax.lax.rsqrt(v1 + 1e-5)
        h1 = h1_ref[pl.ds(t * TB, TB), :]
        a1 = jnp.maximum((h1 - m1) * inv1 * g1_ref[...] + be1_ref[...], 0.0)
        h2 = jnp.dot(a1, W2_ref[...],
                     preferred_element_type=jnp.float32) + b2_ref[...]
        h2_ref[pl.ds(t * TB, TB), :] = h2
        s2_ref[...] += jnp.sum(h2, axis=0, keepdims=True)
        q2_ref[...] += jnp.sum(h2 * h2, axis=0, keepdims=True)

    @pl.when(p == 2)
    def _():
        m2 = s2_ref[...] * inv_b
        v2 = q2_ref[...] * inv_b - m2 * m2
        inv2 = jax.lax.rsqrt(v2 + 1e-5)
        h2 = h2_ref[pl.ds(t * TB, TB), :]
        a2 = jnp.maximum((h2 - m2) * inv2 * g2_ref[...] + be2_ref[...], 0.0)
        y = jnp.sum(a2 * W3_ref[...], axis=1, keepdims=True) + b3_ref[...]
        out_ref[...] = jax.nn.sigmoid(y)


def _tc_mlp(dnn, W1p, b1, g1, be1, W2, b2, g2, be2, W3, b3):
    row = lambda v: v.reshape(1, -1)
    grid = (3, NT)
    out = pl.pallas_call(
        _mlp_body,
        grid=grid,
        in_specs=[
            pl.BlockSpec((TB, KPAD), lambda p, t: (jnp.where(p == 0, t, 0), 0)),
            pl.BlockSpec((KPAD, 64), lambda p, t: (0, 0)),
            pl.BlockSpec((1, 64), lambda p, t: (0, 0)),
            pl.BlockSpec((1, 64), lambda p, t: (0, 0)),
            pl.BlockSpec((1, 64), lambda p, t: (0, 0)),
            pl.BlockSpec((64, 32), lambda p, t: (0, 0)),
            pl.BlockSpec((1, 32), lambda p, t: (0, 0)),
            pl.BlockSpec((1, 32), lambda p, t: (0, 0)),
            pl.BlockSpec((1, 32), lambda p, t: (0, 0)),
            pl.BlockSpec((1, 32), lambda p, t: (0, 0)),
            pl.BlockSpec((1, 1), lambda p, t: (0, 0)),
        ],
        out_specs=pl.BlockSpec((TB, 1), lambda p, t: (t, 0)),
        out_shape=jax.ShapeDtypeStruct((B, 1), jnp.float32),
        scratch_shapes=[
            pltpu.VMEM((B, 64), jnp.float32),
            pltpu.VMEM((B, 32), jnp.float32),
            pltpu.VMEM((1, 64), jnp.float32),
            pltpu.VMEM((1, 64), jnp.float32),
            pltpu.VMEM((1, 32), jnp.float32),
            pltpu.VMEM((1, 32), jnp.float32),
        ],
    )(dnn, W1p, row(b1), row(g1), row(be1), W2, row(b2), row(g2), row(be2),
      W3.reshape(1, -1), b3.reshape(1, 1))
    return out


def kernel(x, emb, W1, b1, g1, be1, W2, b2, g2, be2, W3, b3):
    embflat = emb.reshape(F * F * FD, D)
    xpad = jnp.pad(x, ((0, 0), (0, 32 - F)))
    dnn = _sc_gather_ffm(xpad, jnp.asarray(FSEL), jnp.asarray(CBASE),
                         embflat)
    W1p = jnp.concatenate(
        [W1, jnp.zeros((KPAD - INPUT_DIM, 64), jnp.float32)], axis=0)
    y = _tc_mlp(dnn, W1p, b1, g1, be1, W2, b2, g2, be2, W3, b3)
    return jnp.squeeze(y, axis=1)
